# baseline (speedup 1.0000x reference)
import jax
import jax.numpy as jnp
from jax import lax
from jax.experimental import pallas as pl
from jax.experimental.pallas import tpu as pltpu
from jax.experimental.pallas import tpu_sc as plsc

_NC = 2
_NS = 16
_NW = _NC * _NS
_C = 112
_L = 16
_NBUF = 3
_G = 6


def _sc_spmm(x, edata, wdata, zeros):
    n, d = x.shape
    npad = zeros.shape[0]
    nsup = edata.shape[1]
    nch = nsup * _G
    rows_per_tile = npad // _NS
    nvec = d // _L
    mesh = plsc.VectorSubcoreMesh(core_axis_name="c", subcore_axis_name="s",
                                  num_cores=_NC, num_subcores=_NS)

    def body(x_hbm, e_hbm, w_hbm, z_hbm, out_hbm,
             acc, ebuf, wsup, rows0, rows1, rows2,
             gsem0, gsem1, gsem2, ssem0, ssem1, ssem2,
             isem0, isem1, wsem0, wsem1):
        cid = lax.axis_index("c")
        sid = lax.axis_index("s")
        wid = sid * _NC + cid
        rows = (rows0, rows1, rows2)
        gsem = (gsem0, gsem1, gsem2)
        ssem = (ssem0, ssem1, ssem2)
        isem = (isem0, isem1)
        wsem = (wsem0, wsem1)

        r0 = sid * rows_per_tile
        pltpu.sync_copy(z_hbm.at[pl.ds(r0, rows_per_tile)],
                        acc.at[pl.ds(r0, rows_per_tile)])
        plsc.subcore_barrier()

        def fetch_super(S, sb):
            pltpu.async_copy(e_hbm.at[wid, S], ebuf.at[sb], isem[sb])
            pltpu.async_copy(w_hbm.at[wid, S], wsup.at[sb], wsem[sb])

        def wait_super(S, sb):
            pltpu.make_async_copy(e_hbm.at[wid, S], ebuf.at[sb],
                                  isem[sb]).wait()
            pltpu.make_async_copy(w_hbm.at[wid, S], wsup.at[sb],
                                  wsem[sb]).wait()

        def gather_start(sb, g, slot):
            pltpu.async_copy(x_hbm.at[ebuf.at[sb, g, 0]], rows[slot],
                             gsem[slot])

        def gather_wait(sb, g, slot):
            pltpu.make_async_copy(x_hbm.at[ebuf.at[sb, g, 0]], rows[slot],
                                  gsem[slot]).wait()

        def scat_start(sb, g, slot):
            pltpu.make_async_copy(rows[slot], acc.at[ebuf.at[sb, g, 1]],
                                  ssem[slot]).start(add=True)

        def scat_wait(sb, g, slot):
            pltpu.make_async_copy(rows[slot], acc.at[ebuf.at[sb, g, 1]],
                                  ssem[slot]).wait()

        fetch_super(0, 0)
        wait_super(0, 0)
        gather_start(0, 0, 0)
        gather_start(0, 1, 1)

        def pair(t, carry):
            for sb in range(2):
                S = 2 * t + sb
                for g in range(_G):
                    jj = S * _G + g
                    b = (sb * _G + g) % _NBUF
                    sn = (b + 2) % _NBUF

                    gather_wait(sb, g, b)

                    @plsc.parallel_loop(0, _C // _L)
                    def _(gg):
                        wg = wsup[sb, g, pl.ds(gg * _L, _L)]
                        for l in range(_L):
                            wvec = jnp.full((_L,), wg[l], dtype=jnp.float32)
                            e = gg * _L + l
                            for k in range(nvec):
                                sl = pl.ds(k * _L, _L)
                                rows[b][e, sl] = rows[b][e, sl] * wvec

                    if g == 0:
                        @pl.when(jj >= 1)
                        def _():
                            scat_wait(1 - sb, _G - 1, sn)

                        @pl.when(S + 1 < nsup)
                        def _():
                            fetch_super(S + 1, 1 - sb)
                    else:
                        scat_wait(sb, g - 1, sn)

                    if g == _G - 2:
                        @pl.when(S + 1 < nsup)
                        def _():
                            wait_super(S + 1, 1 - sb)

                    if g < _G - 2:
                        gather_start(sb, g + 2, sn)
                    else:
                        @pl.when(jj + 2 < nch)
                        def _():
                            gather_start(1 - sb, g + 2 - _G, sn)

                    scat_start(sb, g, b)
            return carry

        lax.fori_loop(0, nsup // 2, pair, 0)

        scat_wait((nsup - 1) % 2, _G - 1, (2 * _G - 1) % _NBUF)

        plsc.subcore_barrier()
        pltpu.sync_copy(acc.at[pl.ds(r0, rows_per_tile)],
                        out_hbm.at[cid, pl.ds(r0, rows_per_tile)])

    return pl.kernel(
        body,
        out_type=jax.ShapeDtypeStruct((_NC, npad, d), jnp.float32),
        mesh=mesh,
        scratch_types=[
            pltpu.VMEM_SHARED((npad, d), jnp.float32),
            pltpu.VMEM((2, _G, 2, _C), jnp.int32),
            pltpu.VMEM((2, _G, _C), jnp.float32),
            pltpu.VMEM((_C, d), jnp.float32),
            pltpu.VMEM((_C, d), jnp.float32),
            pltpu.VMEM((_C, d), jnp.float32),
            pltpu.SemaphoreType.DMA,
            pltpu.SemaphoreType.DMA,
            pltpu.SemaphoreType.DMA,
            pltpu.SemaphoreType.DMA,
            pltpu.SemaphoreType.DMA,
            pltpu.SemaphoreType.DMA,
            pltpu.SemaphoreType.DMA,
            pltpu.SemaphoreType.DMA,
            pltpu.SemaphoreType.DMA,
            pltpu.SemaphoreType.DMA,
        ],
    )(x, edata, wdata, zeros)


def _tc_linear(partial, W, b2, n):
    d = partial.shape[2]
    blk = 1000

    def body(p_ref, w_ref, b_ref, o_ref):
        h = p_ref[0] + p_ref[1]
        o_ref[...] = lax.dot_general(
            h, w_ref[...], (((1,), (1,)), ((), ())),
            preferred_element_type=jnp.float32) + b_ref[...]

    return pl.pallas_call(
        body,
        grid=(n // blk,),
        in_specs=[
            pl.BlockSpec((2, blk, d), lambda i: (0, i, 0)),
            pl.BlockSpec((d, d), lambda i: (0, 0)),
            pl.BlockSpec((1, d), lambda i: (0, 0)),
        ],
        out_specs=pl.BlockSpec((blk, d), lambda i: (i, 0)),
        out_shape=jax.ShapeDtypeStruct((n, d), jnp.float32),
    )(partial, W, b2)


def kernel(x, edge_index, edge_weight, W, b):
    n, d = x.shape
    e = edge_index.shape[1]
    quantum = _NW * _C * _G * 2
    ep = quantum * ((e + quantum - 1) // quantum)
    pad = ep - e
    nsup = ep // (_NW * _C * _G)
    src = jnp.concatenate(
        [edge_index[0].astype(jnp.int32), jnp.zeros((pad,), jnp.int32)])
    dst = jnp.concatenate(
        [edge_index[1].astype(jnp.int32), jnp.zeros((pad,), jnp.int32)])
    w = jnp.concatenate(
        [edge_weight.astype(jnp.float32), jnp.zeros((pad,), jnp.float32)])
    edata = jnp.stack([src.reshape(_NW, nsup, _G, _C),
                       dst.reshape(_NW, nsup, _G, _C)], axis=3)
    wdata = w.reshape(_NW, nsup, _G, _C)
    nq = 8 * _NS
    npad = nq * ((n + nq - 1) // nq)
    zeros = jnp.zeros((npad, d), jnp.float32)
    partial = _sc_spmm(x, edata, wdata, zeros)
    return _tc_linear(partial, W, b.reshape(1, d), n)


# speedup vs baseline: 3.2631x; 3.2631x over previous
import jax
import jax.numpy as jnp
from jax import lax
from jax.experimental import pallas as pl
from jax.experimental.pallas import tpu as pltpu
from jax.experimental.pallas import tpu_sc as plsc

_NC = 2
_NS = 16
_NW = _NC * _NS
_C = 112
_L = 16
_RB = 3
_RI = 6


def _sc_spmm(x, edata, wdata, zeros):
    n, d = x.shape
    npad = zeros.shape[0]
    nch = edata.shape[1]
    rows_per_tile = npad // _NS
    nvec = d // _L
    mesh = plsc.VectorSubcoreMesh(core_axis_name="c", subcore_axis_name="s",
                                  num_cores=_NC, num_subcores=_NS)

    def body(x_hbm, e_hbm, w_hbm, z_hbm, out_hbm,
             acc, ebuf, wbuf, rows, gsem, ssem, isem, wsem):
        cid = lax.axis_index("c")
        sid = lax.axis_index("s")
        wid = sid * _NC + cid

        r0 = sid * rows_per_tile
        pltpu.sync_copy(z_hbm.at[pl.ds(r0, rows_per_tile)],
                        acc.at[pl.ds(r0, rows_per_tile)])
        plsc.subcore_barrier()

        def idx_fetch(jj):
            s = lax.rem(jj, _RI)
            pltpu.async_copy(e_hbm.at[wid, jj], ebuf.at[s], isem.at[s])
            pltpu.async_copy(w_hbm.at[wid, jj], wbuf.at[s], wsem.at[s])

        def idx_wait(jj):
            s = lax.rem(jj, _RI)
            pltpu.make_async_copy(e_hbm.at[wid, jj], ebuf.at[s],
                                  isem.at[s]).wait()
            pltpu.make_async_copy(w_hbm.at[wid, jj], wbuf.at[s],
                                  wsem.at[s]).wait()

        def gather_start(jj):
            s = lax.rem(jj, _RI)
            b = lax.rem(jj, _RB)
            pltpu.async_copy(x_hbm.at[ebuf.at[s, 0]], rows.at[b], gsem.at[b])

        def gather_wait(jj):
            s = lax.rem(jj, _RI)
            b = lax.rem(jj, _RB)
            pltpu.make_async_copy(x_hbm.at[ebuf.at[s, 0]], rows.at[b],
                                  gsem.at[b]).wait()

        def scat_start(jj):
            s = lax.rem(jj, _RI)
            b = lax.rem(jj, _RB)
            pltpu.make_async_copy(rows.at[b], acc.at[ebuf.at[s, 1]],
                                  ssem.at[b]).start(add=True)

        def scat_wait(jj):
            s = lax.rem(jj, _RI)
            b = lax.rem(jj, _RB)
            pltpu.make_async_copy(rows.at[b], acc.at[ebuf.at[s, 1]],
                                  ssem.at[b]).wait()

        for jj in range(_RI - 1):
            idx_fetch(jj)
        idx_wait(0)
        idx_wait(1)
        gather_start(0)
        gather_start(1)

        def chunk(jj, carry):
            b = lax.rem(jj, _RB)
            gather_wait(jj)

            @plsc.parallel_loop(0, _C // _L)
            def _(gg):
                s = lax.rem(jj, _RI)
                wg = wbuf[s, pl.ds(gg * _L, _L)]
                for l in range(_L):
                    wvec = jnp.full((_L,), wg[l], dtype=jnp.float32)
                    e = gg * _L + l
                    for k in range(nvec):
                        sl = pl.ds(k * _L, _L)
                        rows[b, e, sl] = rows[b, e, sl] * wvec

            @pl.when(jj >= 1)
            def _():
                scat_wait(jj - 1)

            @pl.when(jj + _RI - 1 < nch)
            def _():
                idx_fetch(jj + _RI - 1)

            @pl.when(jj + 2 < nch)
            def _():
                idx_wait(jj + 2)
                gather_start(jj + 2)

            scat_start(jj)
            return carry

        lax.fori_loop(0, nch, chunk, 0)
        scat_wait(nch - 1)

        plsc.subcore_barrier()
        pltpu.sync_copy(acc.at[pl.ds(r0, rows_per_tile)],
                        out_hbm.at[cid, pl.ds(r0, rows_per_tile)])

    return pl.kernel(
        body,
        out_type=jax.ShapeDtypeStruct((_NC, npad, d), jnp.float32),
        mesh=mesh,
        scratch_types=[
            pltpu.VMEM_SHARED((npad, d), jnp.float32),
            pltpu.VMEM((_RI, 2, _C), jnp.int32),
            pltpu.VMEM((_RI, _C), jnp.float32),
            pltpu.VMEM((_RB, _C, d), jnp.float32),
            pltpu.SemaphoreType.DMA((_RB,)),
            pltpu.SemaphoreType.DMA((_RB,)),
            pltpu.SemaphoreType.DMA((_RI,)),
            pltpu.SemaphoreType.DMA((_RI,)),
        ],
    )(x, edata, wdata, zeros)


def _tc_linear(partial, W, b2, n):
    d = partial.shape[2]
    blk = 1000

    def body(p_ref, w_ref, b_ref, o_ref):
        h = p_ref[0] + p_ref[1]
        o_ref[...] = lax.dot_general(
            h, w_ref[...], (((1,), (1,)), ((), ())),
            preferred_element_type=jnp.float32) + b_ref[...]

    return pl.pallas_call(
        body,
        grid=(n // blk,),
        in_specs=[
            pl.BlockSpec((2, blk, d), lambda i: (0, i, 0)),
            pl.BlockSpec((d, d), lambda i: (0, 0)),
            pl.BlockSpec((1, d), lambda i: (0, 0)),
        ],
        out_specs=pl.BlockSpec((blk, d), lambda i: (i, 0)),
        out_shape=jax.ShapeDtypeStruct((n, d), jnp.float32),
    )(partial, W, b2)


def kernel(x, edge_index, edge_weight, W, b):
    n, d = x.shape
    e = edge_index.shape[1]
    quantum = _NW * _C
    ep = quantum * ((e + quantum - 1) // quantum)
    pad = ep - e
    nch = ep // quantum
    src = jnp.concatenate(
        [edge_index[0].astype(jnp.int32), jnp.zeros((pad,), jnp.int32)])
    dst = jnp.concatenate(
        [edge_index[1].astype(jnp.int32), jnp.zeros((pad,), jnp.int32)])
    w = jnp.concatenate(
        [edge_weight.astype(jnp.float32), jnp.zeros((pad,), jnp.float32)])
    edata = jnp.stack([src.reshape(_NW, nch, _C),
                       dst.reshape(_NW, nch, _C)], axis=2)
    wdata = w.reshape(_NW, nch, _C)
    nq = 8 * _NS
    npad = nq * ((n + nq - 1) // nq)
    zeros = jnp.zeros((npad, d), jnp.float32)
    partial = _sc_spmm(x, edata, wdata, zeros)
    return _tc_linear(partial, W, b.reshape(1, d), n)


# speedup vs baseline: 3.3037x; 1.0124x over previous
import numpy as np
import jax
import jax.numpy as jnp
from jax import lax
from jax.experimental import pallas as pl
from jax.experimental.pallas import tpu as pltpu
from jax.experimental.pallas import tpu_sc as plsc

_NC = 2
_NS = 16
_NW = _NC * _NS
_C = 112
_L = 16
_RB = 3
_RF = 2
_RI = 6
_HC = _C // 2


def _sc_spmm(xb, edata, wdata, zeros):
    n, d = xb.shape
    npad = zeros.shape[0]
    nch = edata.shape[1]
    rows_per_tile = npad // _NS
    mesh = plsc.VectorSubcoreMesh(core_axis_name="c", subcore_axis_name="s",
                                  num_cores=_NC, num_subcores=_NS)

    def body(xb_hbm, e_hbm, w_hbm, z_hbm, out_hbm,
             acc, ebuf, wbuf, rbf, gsem, ssem, isem, wsem):
        cid = lax.axis_index("c")
        sid = lax.axis_index("s")
        wid = sid * _NC + cid

        r0 = sid * rows_per_tile
        pltpu.sync_copy(z_hbm.at[pl.ds(r0, rows_per_tile)],
                        acc.at[pl.ds(r0, rows_per_tile)])
        plsc.subcore_barrier()

        def idx_fetch(jj):
            s = lax.rem(jj, _RI)
            pltpu.async_copy(e_hbm.at[wid, jj], ebuf.at[s], isem.at[s])
            pltpu.async_copy(w_hbm.at[wid, jj], wbuf.at[s], wsem.at[s])

        def idx_wait(jj):
            s = lax.rem(jj, _RI)
            pltpu.make_async_copy(e_hbm.at[wid, jj], ebuf.at[s],
                                  isem.at[s]).wait()
            pltpu.make_async_copy(w_hbm.at[wid, jj], wbuf.at[s],
                                  wsem.at[s]).wait()

        def gather_start(jj):
            s = lax.rem(jj, _RI)
            b = lax.rem(jj, _RB)
            for h in range(2):
                pltpu.async_copy(
                    xb_hbm.at[ebuf.at[s, 0, pl.ds(h * _HC, _HC)]],
                    rbf.at[b, pl.ds(h * _HC, _HC)], gsem.at[b, h])

        def gather_wait(jj):
            s = lax.rem(jj, _RI)
            b = lax.rem(jj, _RB)
            for h in range(2):
                pltpu.make_async_copy(
                    xb_hbm.at[ebuf.at[s, 0, pl.ds(h * _HC, _HC)]],
                    rbf.at[b, pl.ds(h * _HC, _HC)], gsem.at[b, h]).wait()

        def scat_start(jj):
            s = lax.rem(jj, _RI)
            b = lax.rem(jj, _RB)
            pltpu.make_async_copy(rbf.at[b], acc.at[ebuf.at[s, 1]],
                                  ssem.at[b]).start(add=True)

        def scat_wait(jj):
            s = lax.rem(jj, _RI)
            b = lax.rem(jj, _RB)
            pltpu.make_async_copy(rbf.at[b], acc.at[ebuf.at[s, 1]],
                                  ssem.at[b]).wait()

        for jj in range(_RI - 1):
            idx_fetch(jj)
        idx_wait(0)
        idx_wait(1)
        gather_start(0)
        gather_start(1)

        def chunk(jj, carry):
            b = lax.rem(jj, _RB)
            gather_wait(jj)

            @plsc.parallel_loop(0, _C // _L)
            def _(gg):
                s = lax.rem(jj, _RI)
                wg = wbuf[s, pl.ds(gg * _L, _L)]
                for l in range(_L):
                    wvec = jnp.full((_L,), wg[l], dtype=jnp.float32)
                    e = gg * _L + l
                    for k in range(d // _L):
                        sl = pl.ds(k * _L, _L)
                        rbf[b, e, sl] = rbf[b, e, sl] * wvec

            @pl.when(jj >= 1)
            def _():
                scat_wait(jj - 1)

            @pl.when(jj + _RI - 1 < nch)
            def _():
                idx_fetch(jj + _RI - 1)

            @pl.when(jj + 2 < nch)
            def _():
                idx_wait(jj + 2)
                gather_start(jj + 2)

            scat_start(jj)
            return carry

        lax.fori_loop(0, nch, chunk, 0)
        scat_wait(nch - 1)

        plsc.subcore_barrier()
        pltpu.sync_copy(acc.at[pl.ds(r0, rows_per_tile)],
                        out_hbm.at[cid, pl.ds(r0, rows_per_tile)])

    return pl.kernel(
        body,
        out_type=jax.ShapeDtypeStruct((_NC, npad, d), jnp.float32),
        mesh=mesh,
        scratch_types=[
            pltpu.VMEM_SHARED((npad, d), jnp.float32),
            pltpu.VMEM((_RI, 2, _C), jnp.int32),
            pltpu.VMEM((_RI, _C), jnp.float32),
            pltpu.VMEM((_RB, _C, d), jnp.float32),
            pltpu.SemaphoreType.DMA((_RB, 2)),
            pltpu.SemaphoreType.DMA((_RB,)),
            pltpu.SemaphoreType.DMA((_RI,)),
            pltpu.SemaphoreType.DMA((_RI,)),
        ],
    )(xb, edata, wdata, zeros)


def _tc_linear(partial, W, b2, n):
    d = partial.shape[2]
    blk = 1000

    def body(p_ref, w_ref, b_ref, o_ref):
        h = p_ref[0] + p_ref[1]
        o_ref[...] = lax.dot_general(
            h, w_ref[...], (((1,), (1,)), ((), ())),
            preferred_element_type=jnp.float32) + b_ref[...]

    return pl.pallas_call(
        body,
        grid=(n // blk,),
        in_specs=[
            pl.BlockSpec((2, blk, d), lambda i: (0, i, 0)),
            pl.BlockSpec((d, d), lambda i: (0, 0)),
            pl.BlockSpec((1, d), lambda i: (0, 0)),
        ],
        out_specs=pl.BlockSpec((blk, d), lambda i: (i, 0)),
        out_shape=jax.ShapeDtypeStruct((n, d), jnp.float32),
    )(partial, W, b2)


def kernel(x, edge_index, edge_weight, W, b):
    n, d = x.shape
    e = edge_index.shape[1]
    quantum = _NW * _C
    ep = quantum * ((e + quantum - 1) // quantum)
    pad = ep - e
    nch = ep // quantum
    src = jnp.concatenate(
        [edge_index[0].astype(jnp.int32), jnp.zeros((pad,), jnp.int32)])
    dst = jnp.concatenate(
        [edge_index[1].astype(jnp.int32), jnp.zeros((pad,), jnp.int32)])
    w = jnp.concatenate(
        [edge_weight.astype(jnp.float32), jnp.zeros((pad,), jnp.float32)])
    edata = jnp.stack([src.reshape(_NW, nch, _C),
                       dst.reshape(_NW, nch, _C)], axis=2)
    wdata = w.reshape(_NW, nch, _C)
    nq = 8 * _NS
    npad = nq * ((n + nq - 1) // nq)
    zeros = jnp.zeros((npad, d), jnp.float32)
    partial = _sc_spmm(x, edata, wdata, zeros)
    return _tc_linear(partial, W, b.reshape(1, d), n)


# speedup vs baseline: 3.4871x; 1.0555x over previous
import jax
import jax.numpy as jnp
from jax import lax
from jax.experimental import pallas as pl
from jax.experimental.pallas import tpu as pltpu
from jax.experimental.pallas import tpu_sc as plsc

_NC = 2
_NS = 16
_NW = _NC * _NS
_C = 112
_L = 16
_NBUF = 3


def _sc_spmm(x, edata, wdata, zeros):
    n, d = x.shape
    npad = zeros.shape[0]
    nch = edata.shape[1]
    rows_per_tile = npad // _NS
    nvec = d // _L
    mesh = plsc.VectorSubcoreMesh(core_axis_name="c", subcore_axis_name="s",
                                  num_cores=_NC, num_subcores=_NS)

    def body(x_hbm, e_hbm, w_hbm, z_hbm, out_hbm,
             acc, ebuf, wbuf, rows0, rows1, rows2,
             gsem0, gsem1, gsem2, ssem0, ssem1, ssem2):
        cid = lax.axis_index("c")
        sid = lax.axis_index("s")
        wid = sid * _NC + cid
        rows = (rows0, rows1, rows2)
        gsem = (gsem0, gsem1, gsem2)
        ssem = (ssem0, ssem1, ssem2)

        r0 = sid * rows_per_tile
        pltpu.sync_copy(z_hbm.at[pl.ds(r0, rows_per_tile)],
                        acc.at[pl.ds(r0, rows_per_tile)])
        plsc.subcore_barrier()

        def fetch_and_gather(jj, s):
            pltpu.sync_copy(e_hbm.at[wid, jj], ebuf.at[s])
            pltpu.sync_copy(w_hbm.at[wid, jj], wbuf.at[s])
            pltpu.async_copy(x_hbm.at[ebuf.at[s, 0]], rows[s], gsem[s])

        fetch_and_gather(0, 0)
        fetch_and_gather(1, 1)

        def triple(t, carry):
            for b in range(_NBUF):
                jj = _NBUF * t + b
                sn = (b + 2) % _NBUF

                pltpu.make_async_copy(
                    x_hbm.at[ebuf.at[b, 0]], rows[b], gsem[b]).wait()

                @plsc.parallel_loop(0, _C // _L)
                def _(g):
                    wg = wbuf[b, pl.ds(g * _L, _L)]
                    for l in range(_L):
                        wvec = jnp.full((_L,), wg[l], dtype=jnp.float32)
                        e = g * _L + l
                        for k in range(nvec):
                            sl = pl.ds(k * _L, _L)
                            rows[b][e, sl] = rows[b][e, sl] * wvec

                @pl.when(jj >= 1)
                def _():
                    pltpu.make_async_copy(
                        rows[sn], acc.at[ebuf.at[sn, 1]], ssem[sn]).wait()

                @pl.when(jj + 2 < nch)
                def _():
                    fetch_and_gather(jj + 2, sn)

                pltpu.make_async_copy(
                    rows[b], acc.at[ebuf.at[b, 1]], ssem[b]).start(add=True)
            return carry

        lax.fori_loop(0, nch // _NBUF, triple, 0)

        sl_ = (nch - 1) % _NBUF
        pltpu.make_async_copy(
            rows[sl_], acc.at[ebuf.at[sl_, 1]], ssem[sl_]).wait()

        plsc.subcore_barrier()
        pltpu.sync_copy(acc.at[pl.ds(r0, rows_per_tile)],
                        out_hbm.at[cid, pl.ds(r0, rows_per_tile)])

    return pl.kernel(
        body,
        out_type=jax.ShapeDtypeStruct((_NC, npad, d), jnp.float32),
        mesh=mesh,
        scratch_types=[
            pltpu.VMEM_SHARED((npad, d), jnp.float32),
            pltpu.VMEM((_NBUF, 2, _C), jnp.int32),
            pltpu.VMEM((_NBUF, _C), jnp.float32),
            pltpu.VMEM((_C, d), jnp.float32),
            pltpu.VMEM((_C, d), jnp.float32),
            pltpu.VMEM((_C, d), jnp.float32),
            pltpu.SemaphoreType.DMA,
            pltpu.SemaphoreType.DMA,
            pltpu.SemaphoreType.DMA,
            pltpu.SemaphoreType.DMA,
            pltpu.SemaphoreType.DMA,
            pltpu.SemaphoreType.DMA,
        ],
    )(x, edata, wdata, zeros)


def _tc_linear(partial, W, b2, n):
    d = partial.shape[2]
    blk = 1000

    def body(p_ref, w_ref, b_ref, o_ref):
        h = p_ref[0] + p_ref[1]
        o_ref[...] = lax.dot_general(
            h, w_ref[...], (((1,), (1,)), ((), ())),
            preferred_element_type=jnp.float32) + b_ref[...]

    return pl.pallas_call(
        body,
        grid=(n // blk,),
        in_specs=[
            pl.BlockSpec((2, blk, d), lambda i: (0, i, 0)),
            pl.BlockSpec((d, d), lambda i: (0, 0)),
            pl.BlockSpec((1, d), lambda i: (0, 0)),
        ],
        out_specs=pl.BlockSpec((blk, d), lambda i: (i, 0)),
        out_shape=jax.ShapeDtypeStruct((n, d), jnp.float32),
    )(partial, W, b2)


def kernel(x, edge_index, edge_weight, W, b):
    n, d = x.shape
    e = edge_index.shape[1]
    quantum = _NW * _C * _NBUF
    ep = quantum * ((e + quantum - 1) // quantum)
    pad = ep - e
    nch = ep // (_NW * _C)
    src = jnp.concatenate(
        [edge_index[0].astype(jnp.int32), jnp.zeros((pad,), jnp.int32)])
    dst = jnp.concatenate(
        [edge_index[1].astype(jnp.int32), jnp.zeros((pad,), jnp.int32)])
    w = jnp.concatenate(
        [edge_weight.astype(jnp.float32), jnp.zeros((pad,), jnp.float32)])
    edata = jnp.stack([src.reshape(_NW, nch, _C),
                       dst.reshape(_NW, nch, _C)], axis=2)
    wdata = w.reshape(_NW, nch, _C)
    nq = 8 * _NS
    npad = nq * ((n + nq - 1) // nq)
    zeros = jnp.zeros((npad, d), jnp.float32)
    partial = _sc_spmm(x, edata, wdata, zeros)
    return _tc_linear(partial, W, b.reshape(1, d), n)
